# baseline (speedup 1.0000x reference)
import jax
import jax.numpy as jnp
from jax.experimental import pallas as pl

B = 32
S = 4096
T = B * S
N_CLASSES = 512
VEC_LEN = 32
NUM_SAMPLE = 4
CHUNK = 2048
N_CHUNKS = T // CHUNK


def _roll(arr, shift):
    return jnp.roll(arr, shift, axis=1)


def _sq(x, et, v):
    diff = x[:, v:v + 1] - et[v:v + 1, :]
    return diff * diff


def _main_body(x_ref, e_ref, out0_ref, out1_ref, hist_ref, ent_ref):
    xt = x_ref[...]
    x = xt.T
    emb = e_ref[...]
    et = emb.T

    d2 = None
    for g in range(4):
        v0 = 8 * g
        a = _sq(x, et, v0 + 0) + _sq(x, et, v0 + 4)
        b = _sq(x, et, v0 + 2) + _sq(x, et, v0 + 6)
        c = _sq(x, et, v0 + 1) + _sq(x, et, v0 + 5)
        e4 = _sq(x, et, v0 + 3) + _sq(x, et, v0 + 7)
        h8 = (a + b) + (c + e4)
        d2 = h8 if d2 is None else d2 + h8

    d = d2 * jax.lax.rsqrt(d2)

    t = d
    for q in range(1, 8):
        t = t + _roll(d, -8 * q)
    u = t + _roll(t, -4)
    v = u + _roll(u, -2)
    w = v + _roll(v, -1)
    gmean = w * jnp.float32(0.015625)

    iota = jax.lax.broadcasted_iota(jnp.int32, (CHUNK, N_CLASSES), 1)
    inf = jnp.float32(jnp.inf)
    at_group_lane = (iota & 63) == 0
    wm = jnp.where(at_group_lane, gmean, inf)
    mg = jnp.min(wm, axis=1, keepdims=True)
    first_lane = jnp.min(jnp.where(wm == mg, iota, 1024), axis=1, keepdims=True)
    gsel = jax.lax.shift_right_logical(first_lane, 6)

    lane_group = jax.lax.shift_right_logical(iota, 6)
    recip = jnp.float32(1.0) / d
    masked = jnp.where(lane_group == gsel, recip, jnp.float32(0.0))

    work = masked
    ps = []
    onehots = []
    for k in range(NUM_SAMPLE):
        mk = jnp.max(work, axis=1, keepdims=True)
        sel = jnp.min(jnp.where(work == mk, iota, 1024), axis=1, keepdims=True)
        oh = iota == sel
        ps.append(mk)
        onehots.append(oh)
        work = jnp.where(oh, jnp.float32(-1.0), work)

    sabs = ((ps[0] + ps[1]) + ps[2]) + ps[3]
    denom = jnp.maximum(sabs, jnp.float32(1e-12))
    coeff = None
    for k in range(NUM_SAMPLE):
        c = jnp.where(onehots[k], ps[k] / denom, jnp.float32(0.0))
        coeff = c if coeff is None else coeff + c

    output = jnp.dot(coeff, emb,
                     preferred_element_type=jnp.float32,
                     precision=jax.lax.Precision.HIGHEST)

    outt = output.T
    out0_ref[...] = (outt - xt) + xt
    rt = xt - outt
    rsq = rt * rt
    out1_ref[...] = jnp.sum(rsq, axis=0, keepdims=True).reshape(
        1, 1, CHUNK)

    row = jnp.sum(onehots[0].astype(jnp.float32), axis=0, keepdims=True)

    pid = pl.program_id(0) * (S // CHUNK) + pl.program_id(1)

    @pl.when(pid == 0)
    def _init():
        hist_ref[...] = jnp.zeros_like(hist_ref)

    hist_ref[...] += row

    @pl.when(pid == N_CHUNKS - 1)
    def _entropy():
        h = hist_ref[...]
        prob = h / jnp.float32(T)
        safe = jnp.where(h > 0, prob, jnp.float32(1.0))
        plogp = jnp.where(h > 0, prob * jnp.log(safe), jnp.float32(0.0))
        ent_ref[...] = -jnp.sum(plogp, keepdims=True).reshape(1, 1)


@jax.jit
def kernel(x0, embedding0):
    xbv = x0.transpose(0, 3, 2, 1).reshape(B * VEC_LEN, S)
    e = embedding0.reshape(N_CLASSES, VEC_LEN)

    out0, out1, _, ent = pl.pallas_call(
        _main_body,
        grid=(B, S // CHUNK),
        in_specs=[
            pl.BlockSpec((VEC_LEN, CHUNK), lambda b, j: (b, j)),
            pl.BlockSpec((N_CLASSES, VEC_LEN), lambda b, j: (0, 0)),
        ],
        out_specs=[
            pl.BlockSpec((VEC_LEN, CHUNK), lambda b, j: (b, j)),
            pl.BlockSpec((1, 1, CHUNK), lambda b, j: (b, 0, j)),
            pl.BlockSpec((1, N_CLASSES), lambda b, j: (0, 0)),
            pl.BlockSpec((1, 1), lambda b, j: (0, 0)),
        ],
        out_shape=[
            jax.ShapeDtypeStruct((B * VEC_LEN, S), jnp.float32),
            jax.ShapeDtypeStruct((B, 1, S), jnp.float32),
            jax.ShapeDtypeStruct((1, N_CLASSES), jnp.float32),
            jax.ShapeDtypeStruct((1, 1), jnp.float32),
        ],
    )(xbv, e)

    out0_f = out0.reshape(B, VEC_LEN, 1, S).transpose(0, 3, 2, 1)
    out1_f = out1.transpose(0, 2, 1)
    return (out0_f, out1_f, out1_f, ent.reshape(()))


# speedup vs baseline: 1.2176x; 1.2176x over previous
import jax
import jax.numpy as jnp
from jax.experimental import pallas as pl

B = 32
S = 4096
T = B * S
N_CLASSES = 512
VEC_LEN = 32
NUM_SAMPLE = 4
CHUNK = 1024
N_CHUNKS = T // CHUNK


def _roll(arr, shift):
    return jnp.roll(arr, shift, axis=1)


def _sq(x, et, v):
    diff = x[:, v:v + 1] - et[v:v + 1, :]
    return diff * diff


def _main_body(x_ref, e_ref, out0_ref, out1_ref, hist_ref, ent_ref):
    xt = x_ref[...]
    x = xt.T
    emb = e_ref[...]
    et = emb.T

    d2 = None
    for g in range(4):
        v0 = 8 * g
        a = _sq(x, et, v0 + 0) + _sq(x, et, v0 + 4)
        b = _sq(x, et, v0 + 2) + _sq(x, et, v0 + 6)
        c = _sq(x, et, v0 + 1) + _sq(x, et, v0 + 5)
        e4 = _sq(x, et, v0 + 3) + _sq(x, et, v0 + 7)
        h8 = (a + b) + (c + e4)
        d2 = h8 if d2 is None else d2 + h8

    d = d2 * jax.lax.rsqrt(d2)

    t = d
    for q in range(1, 8):
        t = t + _roll(d, -8 * q)
    u = t + _roll(t, -4)
    v = u + _roll(u, -2)
    w = v + _roll(v, -1)
    gmean = w * jnp.float32(0.015625)

    iota = jax.lax.broadcasted_iota(jnp.int32, (CHUNK, N_CLASSES), 1)
    inf = jnp.float32(jnp.inf)
    at_group_lane = (iota & 63) == 0
    wm = jnp.where(at_group_lane, gmean, inf)
    mg = jnp.min(wm, axis=1, keepdims=True)
    first_lane = jnp.min(jnp.where(wm == mg, iota, 1024), axis=1, keepdims=True)
    gsel = jax.lax.shift_right_logical(first_lane, 6)

    lane_group = jax.lax.shift_right_logical(iota, 6)
    recip = jnp.float32(1.0) / d
    masked = jnp.where(lane_group == gsel, recip, jnp.float32(0.0))

    r1 = masked + _roll(masked, -256)
    r2 = r1 + _roll(r1, -128)
    r3 = r2 + _roll(r2, -64)
    comp = r3[:, :64]

    iota64 = jax.lax.broadcasted_iota(jnp.int32, (CHUNK, 64), 1)
    work = comp
    ps = []
    onehots = []
    for k in range(NUM_SAMPLE):
        mk = jnp.max(work, axis=1, keepdims=True)
        sel = jnp.min(jnp.where(work == mk, iota64, 64), axis=1, keepdims=True)
        oh = iota64 == sel
        ps.append(mk)
        onehots.append(oh)
        work = jnp.where(oh, jnp.float32(-1.0), work)

    sabs = ((ps[0] + ps[1]) + ps[2]) + ps[3]
    denom = jnp.maximum(sabs, jnp.float32(1e-12))
    coeffc = None
    for k in range(NUM_SAMPLE):
        c = jnp.where(onehots[k], ps[k] / denom, jnp.float32(0.0))
        coeffc = c if coeffc is None else coeffc + c

    coeff = jnp.where(lane_group == gsel,
                      jnp.concatenate([coeffc] * 8, axis=1),
                      jnp.float32(0.0))

    output = jnp.dot(coeff, emb,
                     preferred_element_type=jnp.float32,
                     precision=jax.lax.Precision.HIGHEST)

    outt = output.T
    out0_ref[...] = (outt - xt) + xt
    rt = xt - outt
    rsq = rt * rt
    out1_ref[...] = jnp.sum(rsq, axis=0, keepdims=True).reshape(
        1, 1, CHUNK)

    oh0 = onehots[0].astype(jnp.float32)
    oh0_full = jnp.where(lane_group == gsel,
                         jnp.concatenate([oh0] * 8, axis=1),
                         jnp.float32(0.0))
    row = jnp.sum(oh0_full, axis=0, keepdims=True)

    pid = pl.program_id(0) * (S // CHUNK) + pl.program_id(1)

    @pl.when(pid == 0)
    def _init():
        hist_ref[...] = jnp.zeros_like(hist_ref)

    hist_ref[...] += row

    @pl.when(pid == N_CHUNKS - 1)
    def _entropy():
        h = hist_ref[...]
        prob = h / jnp.float32(T)
        safe = jnp.where(h > 0, prob, jnp.float32(1.0))
        plogp = jnp.where(h > 0, prob * jnp.log(safe), jnp.float32(0.0))
        ent_ref[...] = -jnp.sum(plogp, keepdims=True).reshape(1, 1)


@jax.jit
def kernel(x0, embedding0):
    xbv = x0.transpose(0, 3, 2, 1).reshape(B * VEC_LEN, S)
    e = embedding0.reshape(N_CLASSES, VEC_LEN)

    out0, out1, _, ent = pl.pallas_call(
        _main_body,
        grid=(B, S // CHUNK),
        in_specs=[
            pl.BlockSpec((VEC_LEN, CHUNK), lambda b, j: (b, j)),
            pl.BlockSpec((N_CLASSES, VEC_LEN), lambda b, j: (0, 0)),
        ],
        out_specs=[
            pl.BlockSpec((VEC_LEN, CHUNK), lambda b, j: (b, j)),
            pl.BlockSpec((1, 1, CHUNK), lambda b, j: (b, 0, j)),
            pl.BlockSpec((1, N_CLASSES), lambda b, j: (0, 0)),
            pl.BlockSpec((1, 1), lambda b, j: (0, 0)),
        ],
        out_shape=[
            jax.ShapeDtypeStruct((B * VEC_LEN, S), jnp.float32),
            jax.ShapeDtypeStruct((B, 1, S), jnp.float32),
            jax.ShapeDtypeStruct((1, N_CLASSES), jnp.float32),
            jax.ShapeDtypeStruct((1, 1), jnp.float32),
        ],
    )(xbv, e)

    out0_f = out0.reshape(B, VEC_LEN, 1, S).transpose(0, 3, 2, 1)
    out1_f = out1.transpose(0, 2, 1)
    return (out0_f, out1_f, out1_f, ent.reshape(()))
